# Initial kernel scaffold; baseline (speedup 1.0000x reference)
import jax
import jax.numpy as jnp
from jax.experimental import pallas as pl


def kernel(x, weight):
    raise NotImplementedError("write your pallas kernel here")


# speedup vs baseline: 10.8477x; 10.8477x over previous
import functools

import jax
import jax.numpy as jnp
from jax.experimental import pallas as pl
from jax.experimental.pallas import tpu as pltpu

_TOPK = 8
_N_GROUPS = 8
_TOPK_GROUPS = 4
_ROUTE_SCALE = 2.5
_NEG = -1e30


def _gate_kernel(x_ref, w_ref, wout_ref, iout_ref):
    tb = x_ref.shape[0]
    z = jax.lax.dot_general(
        w_ref[...], x_ref[...],
        dimension_numbers=(((1,), (1,)), ((), ())),
        preferred_element_type=jnp.float32)
    s = 1.0 / (1.0 + jnp.exp(-z))

    g = s.reshape(_N_GROUPS, 8, tb)
    m1 = jnp.max(g, axis=1)
    eq = g == m1[:, None, :]
    cnt = jnp.sum(eq.astype(jnp.float32), axis=1)
    m2 = jnp.where(cnt >= 2.0, m1,
                   jnp.max(jnp.where(eq, _NEG, g), axis=1))
    gs = m1 + m2

    p = gs[None, :, :]
    q = gs[:, None, :]
    gp = jax.lax.broadcasted_iota(jnp.int32, (_N_GROUPS, _N_GROUPS, tb), 1)
    gq = jax.lax.broadcasted_iota(jnp.int32, (_N_GROUPS, _N_GROUPS, tb), 0)
    beats = (p > q) | ((p == q) & (gp < gq))
    rank = jnp.sum(beats.astype(jnp.int32), axis=1)
    selg = rank < _TOPK_GROUPS
    sel = jnp.broadcast_to(selg[:, None, :], (_N_GROUPS, 8, tb))
    masked = jnp.where(sel.reshape(64, tb), s, _NEG)

    eio = jax.lax.broadcasted_iota(jnp.int32, (64, tb), 0)
    vals, idxs = [], []
    for _ in range(_TOPK):
        m = jnp.max(masked, axis=0, keepdims=True)
        cand = jnp.where(masked == m, eio, 64)
        bi = jnp.min(cand, axis=0, keepdims=True)
        vals.append(m)
        idxs.append(bi)
        masked = jnp.where(eio == bi, _NEG, masked)

    w8 = jnp.concatenate(vals, axis=0)
    i8 = jnp.concatenate(idxs, axis=0)
    wsum = jnp.sum(w8, axis=0, keepdims=True)
    wout_ref[...] = w8 * (_ROUTE_SCALE / wsum)
    iout_ref[...] = i8


@functools.partial(jax.jit, static_argnames=())
def kernel(x, weight):
    t, d = x.shape
    e = weight.shape[0]
    tb = 512
    if t % tb != 0:
        tb = 256 if t % 256 == 0 else t
    grid = (t // tb,)
    w8, i8 = pl.pallas_call(
        _gate_kernel,
        grid=grid,
        in_specs=[
            pl.BlockSpec((tb, d), lambda i: (i, 0)),
            pl.BlockSpec((e, d), lambda i: (0, 0)),
        ],
        out_specs=[
            pl.BlockSpec((_TOPK, tb), lambda i: (0, i)),
            pl.BlockSpec((_TOPK, tb), lambda i: (0, i)),
        ],
        out_shape=[
            jax.ShapeDtypeStruct((_TOPK, t), jnp.float32),
            jax.ShapeDtypeStruct((_TOPK, t), jnp.int32),
        ],
        compiler_params=pltpu.CompilerParams(
            dimension_semantics=("parallel",)),
    )(x, weight)
    return w8.T.astype(x.dtype), i8.T


# speedup vs baseline: 12.1244x; 1.1177x over previous
import functools

import jax
import jax.numpy as jnp
from jax.experimental import pallas as pl
from jax.experimental.pallas import tpu as pltpu

_TOPK = 8
_N_GROUPS = 8
_TOPK_GROUPS = 4
_ROUTE_SCALE = 2.5
_NEG = -1e30


def _gate_kernel(x_ref, w_ref, wout_ref, iout_ref):
    tb = x_ref.shape[0]
    z = jax.lax.dot_general(
        w_ref[...], x_ref[...],
        dimension_numbers=(((1,), (1,)), ((), ())),
        preferred_element_type=jnp.float32)
    s = 1.0 / (1.0 + jnp.exp(-z))

    g = s.reshape(_N_GROUPS, 8, tb)
    m1 = jnp.max(g, axis=1)
    eq = g == m1[:, None, :]
    cnt = jnp.sum(eq.astype(jnp.float32), axis=1)
    m2 = jnp.where(cnt >= 2.0, m1,
                   jnp.max(jnp.where(eq, _NEG, g), axis=1))
    gs = m1 + m2

    giota = jax.lax.broadcasted_iota(jnp.int32, (_N_GROUPS, tb), 0)
    selg = giota >= _N_GROUPS
    for _ in range(_TOPK_GROUPS):
        gm = jnp.max(gs, axis=0, keepdims=True)
        bi = jnp.min(jnp.where(gs == gm, giota, _N_GROUPS),
                     axis=0, keepdims=True)
        hit = giota == bi
        selg = selg | hit
        gs = jnp.where(hit, _NEG, gs)
    sel = jnp.broadcast_to(selg[:, None, :], (_N_GROUPS, 8, tb))
    masked = jnp.where(sel.reshape(64, tb), s, _NEG)

    eio = jax.lax.broadcasted_iota(jnp.int32, (64, tb), 0)
    wsum = jnp.zeros((1, tb), jnp.float32)
    for r in range(_TOPK):
        m = jnp.max(masked, axis=0, keepdims=True)
        bi = jnp.min(jnp.where(masked == m, eio, 64),
                     axis=0, keepdims=True)
        wout_ref[pl.ds(r, 1), :] = m
        iout_ref[pl.ds(r, 1), :] = bi
        wsum = wsum + m
        masked = jnp.where(eio == bi, _NEG, masked)

    wout_ref[...] = wout_ref[...] * (_ROUTE_SCALE / wsum)


@functools.partial(jax.jit, static_argnames=())
def kernel(x, weight):
    t, d = x.shape
    e = weight.shape[0]
    tb = 512
    if t % tb != 0:
        tb = 256 if t % 256 == 0 else t
    grid = (t // tb,)
    w8, i8 = pl.pallas_call(
        _gate_kernel,
        grid=grid,
        in_specs=[
            pl.BlockSpec((tb, d), lambda i: (i, 0)),
            pl.BlockSpec((e, d), lambda i: (0, 0)),
        ],
        out_specs=[
            pl.BlockSpec((_TOPK, tb), lambda i: (0, i)),
            pl.BlockSpec((_TOPK, tb), lambda i: (0, i)),
        ],
        out_shape=[
            jax.ShapeDtypeStruct((_TOPK, t), jnp.float32),
            jax.ShapeDtypeStruct((_TOPK, t), jnp.int32),
        ],
        compiler_params=pltpu.CompilerParams(
            dimension_semantics=("parallel",)),
    )(x, weight)
    return w8.T.astype(x.dtype), i8.T
